# baseline (speedup 1.0000x reference)
import functools

import jax
import jax.numpy as jnp
from jax.experimental import pallas as pl
from jax.experimental.pallas import tpu as pltpu

H = 1024
FF = 2816
E = 8
FF_TILE = 256
N_FT = FF // FF_TILE


def _moe_kernel(x_ref, gate_ref, w1_ref, w2_ref, w3_ref,
                out_ref, logits_ref, coef_ref):
    e = pl.program_id(0)
    f = pl.program_id(1)

    @pl.when((e == 0) & (f == 0))
    def _router():
        x = x_ref[...]
        logits = jnp.dot(x, gate_ref[...], preferred_element_type=jnp.float32)
        logits_ref[...] = logits
        idx = jax.lax.broadcasted_iota(jnp.int32, logits.shape, 1)
        v1 = jnp.max(logits, axis=1, keepdims=True)
        i1 = jnp.min(jnp.where(logits == v1, idx, E), axis=1, keepdims=True)
        oh1 = idx == i1
        masked = jnp.where(oh1, -jnp.inf, logits)
        v2 = jnp.max(masked, axis=1, keepdims=True)
        i2 = jnp.min(jnp.where(masked == v2, idx, E), axis=1, keepdims=True)
        oh2 = idx == i2
        p1 = 1.0 / (1.0 + jnp.exp(v2 - v1))
        p2 = 1.0 - p1
        coef_ref[...] = jnp.where(oh1, p1, 0.0) + jnp.where(oh2, p2, 0.0)
        out_ref[...] = jnp.zeros_like(out_ref)

    x = x_ref[...].astype(jnp.bfloat16)
    w1b = w1_ref[0].astype(jnp.bfloat16)
    w3b = w3_ref[0].astype(jnp.bfloat16)
    w2b = w2_ref[0].astype(jnp.bfloat16)
    h = jax.nn.silu(jnp.dot(x, w1b, preferred_element_type=jnp.float32))
    h = h * jnp.dot(x, w3b, preferred_element_type=jnp.float32)
    contrib = jnp.dot(h.astype(jnp.bfloat16), w2b,
                      preferred_element_type=jnp.float32)
    c = coef_ref[...]
    lane = jax.lax.broadcasted_iota(jnp.int32, c.shape, 1)
    coef = jnp.sum(jnp.where(lane == e, c, 0.0), axis=1, keepdims=True)
    out_ref[...] += coef * contrib


@jax.jit
def kernel(hidden_states, gate_w, w1, w2, w3):
    B, S, _ = hidden_states.shape
    T = B * S
    x = hidden_states.reshape(T, H)

    grid = (E, N_FT)
    out, logits = pl.pallas_call(
        _moe_kernel,
        grid=grid,
        in_specs=[
            pl.BlockSpec((T, H), lambda e, f: (0, 0)),
            pl.BlockSpec((H, E), lambda e, f: (0, 0)),
            pl.BlockSpec((1, H, FF_TILE), lambda e, f: (e, 0, f)),
            pl.BlockSpec((1, FF_TILE, H), lambda e, f: (e, f, 0)),
            pl.BlockSpec((1, H, FF_TILE), lambda e, f: (e, 0, f)),
        ],
        out_specs=[
            pl.BlockSpec((T, H), lambda e, f: (0, 0)),
            pl.BlockSpec((T, E), lambda e, f: (0, 0)),
        ],
        out_shape=[
            jax.ShapeDtypeStruct((T, H), jnp.float32),
            jax.ShapeDtypeStruct((T, E), jnp.float32),
        ],
        scratch_shapes=[pltpu.VMEM((T, E), jnp.float32)],
    )(x, gate_w, w1, w2, w3)

    return out.reshape(B, S, H), logits.reshape(B, S, E)


# speedup vs baseline: 1.3368x; 1.3368x over previous
import jax
import jax.numpy as jnp
from jax.experimental import pallas as pl
from jax.experimental.pallas import tpu as pltpu

H = 1024
FF = 2816
E = 8
FF_TILE = 1408
N_FT = FF // FF_TILE
N_STEPS = E * N_FT


def _moe_kernel(x_ref, gate_ref, w1_hbm, w2_hbm, w3_hbm,
                out_ref, logits_ref,
                w1_buf, w2_buf, w3_buf, coef_ref, sems):
    def issue(step, slot):
        e, f = step // N_FT, step % N_FT
        pltpu.make_async_copy(
            w1_hbm.at[e, :, pl.ds(f * FF_TILE, FF_TILE)],
            w1_buf.at[slot], sems.at[0, slot]).start()
        pltpu.make_async_copy(
            w2_hbm.at[e, pl.ds(f * FF_TILE, FF_TILE), :],
            w2_buf.at[slot], sems.at[1, slot]).start()
        pltpu.make_async_copy(
            w3_hbm.at[e, :, pl.ds(f * FF_TILE, FF_TILE)],
            w3_buf.at[slot], sems.at[2, slot]).start()

    issue(0, 0)
    issue(1, 1)

    xf = x_ref[...]
    logits = jnp.dot(xf, gate_ref[...], preferred_element_type=jnp.float32)
    logits_ref[...] = logits
    idx = jax.lax.broadcasted_iota(jnp.int32, logits.shape, 1)
    v1 = jnp.max(logits, axis=1, keepdims=True)
    i1 = jnp.min(jnp.where(logits == v1, idx, E), axis=1, keepdims=True)
    oh1 = idx == i1
    masked = jnp.where(oh1, -jnp.inf, logits)
    v2 = jnp.max(masked, axis=1, keepdims=True)
    i2 = jnp.min(jnp.where(masked == v2, idx, E), axis=1, keepdims=True)
    oh2 = idx == i2
    p1 = 1.0 / (1.0 + jnp.exp(v2 - v1))
    coef = jnp.where(oh1, p1, 0.0) + jnp.where(oh2, 1.0 - p1, 0.0)

    x = xf.astype(jnp.bfloat16)
    acc = jnp.zeros_like(out_ref)

    for step in range(N_STEPS):
        slot = step % 2
        e = step // N_FT
        pltpu.make_async_copy(
            w1_hbm.at[0, :, pl.ds(0, FF_TILE)],
            w1_buf.at[slot], sems.at[0, slot]).wait()
        pltpu.make_async_copy(
            w2_hbm.at[0, pl.ds(0, FF_TILE), :],
            w2_buf.at[slot], sems.at[1, slot]).wait()
        pltpu.make_async_copy(
            w3_hbm.at[0, :, pl.ds(0, FF_TILE)],
            w3_buf.at[slot], sems.at[2, slot]).wait()

        w1b = w1_buf[slot].astype(jnp.bfloat16)
        w3b = w3_buf[slot].astype(jnp.bfloat16)
        w2b = w2_buf[slot].astype(jnp.bfloat16)
        h = jax.nn.silu(jnp.dot(x, w1b, preferred_element_type=jnp.float32))
        h = h * jnp.dot(x, w3b, preferred_element_type=jnp.float32)
        contrib = jnp.dot(h.astype(jnp.bfloat16), w2b,
                          preferred_element_type=jnp.float32)
        ce = coef[:, e][:, None]
        acc = acc + ce * contrib

        if step + 2 < N_STEPS:
            issue(step + 2, slot)

    out_ref[...] = acc


@jax.jit
def kernel(hidden_states, gate_w, w1, w2, w3):
    B, S, _ = hidden_states.shape
    T = B * S
    x = hidden_states.reshape(T, H)

    out, logits = pl.pallas_call(
        _moe_kernel,
        in_specs=[
            pl.BlockSpec(memory_space=pltpu.VMEM),
            pl.BlockSpec(memory_space=pltpu.VMEM),
            pl.BlockSpec(memory_space=pl.ANY),
            pl.BlockSpec(memory_space=pl.ANY),
            pl.BlockSpec(memory_space=pl.ANY),
        ],
        out_specs=[
            pl.BlockSpec(memory_space=pltpu.VMEM),
            pl.BlockSpec(memory_space=pltpu.VMEM),
        ],
        out_shape=[
            jax.ShapeDtypeStruct((T, H), jnp.float32),
            jax.ShapeDtypeStruct((T, E), jnp.float32),
        ],
        scratch_shapes=[
            pltpu.VMEM((2, H, FF_TILE), jnp.float32),
            pltpu.VMEM((2, FF_TILE, H), jnp.float32),
            pltpu.VMEM((2, H, FF_TILE), jnp.float32),
            pltpu.VMEM((T, E), jnp.float32),
            pltpu.SemaphoreType.DMA((3, 2)),
        ],
    )(x, gate_w, w1, w2, w3)

    return out.reshape(B, S, H), logits.reshape(B, S, E)
